# baseline (speedup 1.0000x reference)
import functools

import jax
import jax.numpy as jnp
from jax import lax
from jax.experimental import pallas as pl
from jax.experimental.pallas import tpu as pltpu
from jax.experimental.pallas import tpu_sc as plsc

N = 10000
E = 160000
D = 256
DE = 16
H = 8
C = 32

NC = 2
NS = 16
NW = NC * NS

NPAD = 10240
ZR = NPAD // NS
TRASH = N

CB = 128
EP = 163840
EPT = EP // NW
EPS = EP // NS

_mesh = plsc.VectorSubcoreMesh(core_axis_name="c", subcore_axis_name="s",
                               num_cores=NC, num_subcores=NS)


@functools.partial(
    pl.kernel,
    out_type=jax.ShapeDtypeStruct((NC, NPAD, 2 * DE), jnp.float32),
    mesh=_mesh,
    scratch_types=[
        pltpu.VMEM((CB,), jnp.int32),
        pltpu.VMEM((CB, DE), jnp.float32),
        pltpu.VMEM((CB, 128), jnp.float32),
        pltpu.VMEM((32, 128), jnp.float32),
        pltpu.VMEM((32, 2 * DE), jnp.float32),
        pltpu.VMEM_SHARED((NPAD, 128), jnp.float32),
    ],
)
def _p1_attr_sums(dst_hbm, ea_hbm, z128_hbm,
                  sc_out,
                  idx_v, ea_v, wide_v, rd_v, pk_v, acc_sh):
    c = lax.axis_index("c")
    s = lax.axis_index("s")
    wid = s * NC + c
    pltpu.sync_copy(z128_hbm.at[pl.ds(s * ZR, ZR)], acc_sh.at[pl.ds(s * ZR, ZR)])
    zv = jnp.zeros((16,), jnp.float32)
    onev = jnp.where(lax.iota(jnp.int32, 16) == 0, 1.0, 0.0)

    def zrow(r, carry):
        for k in range(8):
            wide_v[r, pl.ds(16 * k, 16)] = zv
        return carry

    lax.fori_loop(0, CB, zrow, 0)
    plsc.subcore_barrier()

    def body(i, carry):
        base = wid * EPT + i * CB
        pltpu.sync_copy(dst_hbm.at[pl.ds(base, CB)], idx_v)
        pltpu.sync_copy(ea_hbm.at[pl.ds(base, CB)], ea_v)

        def fill(r, carry2):
            wide_v[r, pl.ds(0, 16)] = ea_v[r, :]
            wide_v[r, pl.ds(16, 16)] = onev
            return carry2

        lax.fori_loop(0, CB, fill, 0)
        pltpu.sync_copy(wide_v, acc_sh.at[idx_v], add=True)
        return carry

    lax.fori_loop(0, EPT // CB, body, 0)
    plsc.subcore_barrier()

    def rdout(t, carry):
        row0 = s * ZR + t * 32
        pltpu.sync_copy(acc_sh.at[pl.ds(row0, 32)], rd_v)

        def pack(r, carry2):
            pk_v[r, pl.ds(0, 16)] = rd_v[r, pl.ds(0, 16)]
            pk_v[r, pl.ds(16, 16)] = rd_v[r, pl.ds(16, 16)]
            return carry2

        lax.fori_loop(0, 32, pack, 0)
        pltpu.sync_copy(pk_v, sc_out.at[c, pl.ds(row0, 32)])
        return carry

    lax.fori_loop(0, ZR // 32, rdout, 0)


ITER = EPS // CB
SG = 2
ITER2 = EPS // (SG * CB)
NCHUNK = EP // CB


def _p3_one_stream(idx_hbm, table_hbm, out_hbm, s, idx_v, buf, g0, g1, w0, w1):
    base0 = s * EPS
    pltpu.sync_copy(idx_hbm.at[pl.ds(base0, EPS)], idx_v)
    b0 = buf.at[0]
    b1 = buf.at[1]

    def gather(i, b, sem):
        for k in range(SG):
            pltpu.async_copy(
                table_hbm.at[idx_v.at[pl.ds((i * SG + k) * CB, CB)]],
                b.at[k], sem)

    def wb(i, b, sem):
        return pltpu.async_copy(b, out_hbm.at[pl.ds(s * ITER2 * SG + i * SG, SG)], sem)

    def wait_g(b, sem):
        pltpu.make_async_copy(out_hbm.at[pl.ds(0, SG)], b, sem).wait()

    def wait_w(b, sem):
        pltpu.make_async_copy(b, out_hbm.at[pl.ds(0, SG)], sem).wait()

    gather(0, b0, g0)
    J = ITER2 // 2

    def body(j, carry):
        i = 2 * j
        wait_g(b0, g0)
        wb(i, b0, w0)

        @pl.when(j > 0)
        def _():
            wait_w(b1, w1)

        gather(i + 1, b1, g1)
        wait_g(b1, g1)
        wb(i + 1, b1, w1)

        @pl.when(j < J - 1)
        def _():
            wait_w(b0, w0)
            gather(i + 2, b0, g0)

        return carry

    lax.fori_loop(0, J, body, 0)
    wait_w(b0, w0)
    wait_w(b1, w1)


@functools.partial(
    pl.kernel,
    out_type=[
        jax.ShapeDtypeStruct((NCHUNK, CB, D // 2), jnp.int32),
        jax.ShapeDtypeStruct((NCHUNK, CB, D // 2), jnp.int32),
    ],
    mesh=_mesh,
    scratch_types=[
        pltpu.VMEM((EPS,), jnp.int32),
        pltpu.VMEM((2, SG, CB, D // 2), jnp.int32),
        pltpu.SemaphoreType.DMA,
        pltpu.SemaphoreType.DMA,
        pltpu.SemaphoreType.DMA,
        pltpu.SemaphoreType.DMA,
    ],
)
def _p3_gather(src_hbm, dst_hbm, xlb_hbm, xrb_hbm,
               gl_out, gr_out,
               idx_v, buf, g0, g1, w0, w1):
    c = lax.axis_index("c")
    s = lax.axis_index("s")

    @pl.when(c == 0)
    def _():
        _p3_one_stream(src_hbm, xlb_hbm, gl_out, s, idx_v, buf, g0, g1, w0, w1)

    @pl.when(c == 1)
    def _():
        _p3_one_stream(dst_hbm, xrb_hbm, gr_out, s, idx_v, buf, g0, g1, w0, w1)


NP5 = 10112
ZR5 = NP5 // NS


@functools.partial(
    pl.kernel,
    out_type=jax.ShapeDtypeStruct((NC, NPAD, D // 2), jnp.float32),
    mesh=_mesh,
    scratch_types=[
        pltpu.VMEM((ITER, CB), jnp.int32),
        pltpu.VMEM((2, CB, D // 2), jnp.float32),
        pltpu.VMEM_SHARED((NP5, D // 2), jnp.float32),
        pltpu.SemaphoreType.DMA,
        pltpu.SemaphoreType.DMA,
        pltpu.SemaphoreType.DMA,
        pltpu.SemaphoreType.DMA,
    ],
)
def _p5_scatter(dst_r_hbm, y_hbm, z128_hbm,
                num_out,
                idx_v, buf, acc_sh, l0, l1, a0, a1):
    c = lax.axis_index("c")
    s = lax.axis_index("s")
    pltpu.sync_copy(z128_hbm.at[pl.ds(s * ZR5, ZR5)], acc_sh.at[pl.ds(s * ZR5, ZR5)])
    pltpu.sync_copy(dst_r_hbm.at[s], idx_v)
    plsc.subcore_barrier()
    b0 = buf.at[0]
    b1 = buf.at[1]
    base0 = s * EPS

    def load(i, b, sem):
        return pltpu.async_copy(y_hbm.at[c, pl.ds(base0 + i * CB, CB)], b, sem)

    def scat(i, b, sem):
        return pltpu.async_copy(b, acc_sh.at[idx_v.at[i]], sem, add=True)

    def wait_l(b, sem):
        pltpu.make_async_copy(y_hbm.at[c, pl.ds(0, CB)], b, sem).wait()

    def wait_a(b, sem):
        pltpu.make_async_copy(b, acc_sh.at[pl.ds(0, CB)], sem).wait()

    load(0, b0, l0)
    J = ITER // 2

    def body(j, carry):
        i = 2 * j
        wait_l(b0, l0)
        scat(i, b0, a0)

        @pl.when(j > 0)
        def _():
            wait_a(b1, a1)

        load(i + 1, b1, l1)
        wait_l(b1, l1)
        scat(i + 1, b1, a1)

        @pl.when(j < J - 1)
        def _():
            wait_a(b0, a0)
            load(i + 2, b0, l0)

        return carry

    lax.fori_loop(0, J, body, 0)
    wait_a(b0, a0)
    wait_a(b1, a1)
    plsc.subcore_barrier()
    pltpu.sync_copy(acc_sh.at[pl.ds(s * ZR5, ZR5)], num_out.at[c, pl.ds(s * ZR5, ZR5)])


@functools.partial(
    pl.kernel,
    out_type=jax.ShapeDtypeStruct((NC, NPAD, DE), jnp.float32),
    mesh=_mesh,
    scratch_types=[
        pltpu.VMEM((CB,), jnp.int32),
        pltpu.VMEM((CB, DE), jnp.float32),
        pltpu.VMEM((CB, 128), jnp.float32),
        pltpu.VMEM((32, 128), jnp.float32),
        pltpu.VMEM((32, DE), jnp.float32),
        pltpu.VMEM_SHARED((NPAD, 128), jnp.float32),
    ],
)
def _p5b_den_scatter(dst_hbm, exz_hbm, z128_hbm,
                     den_out,
                     idx_v, ex_v, wide_v, rd_v, pk_v, acc_sh):
    c = lax.axis_index("c")
    s = lax.axis_index("s")
    wid = s * NC + c
    pltpu.sync_copy(z128_hbm.at[pl.ds(s * ZR, ZR)], acc_sh.at[pl.ds(s * ZR, ZR)])
    zv = jnp.zeros((16,), jnp.float32)

    def zrow(r, carry):
        for k in range(8):
            wide_v[r, pl.ds(16 * k, 16)] = zv
        return carry

    lax.fori_loop(0, CB, zrow, 0)
    plsc.subcore_barrier()

    def body(i, carry):
        base = wid * EPT + i * CB
        pltpu.sync_copy(dst_hbm.at[pl.ds(base, CB)], idx_v)
        pltpu.sync_copy(exz_hbm.at[pl.ds(base, CB)], ex_v)

        def fill(r, carry2):
            wide_v[r, pl.ds(0, 16)] = ex_v[r, :]
            return carry2

        lax.fori_loop(0, CB, fill, 0)
        pltpu.sync_copy(wide_v, acc_sh.at[idx_v], add=True)
        return carry

    lax.fori_loop(0, EPT // CB, body, 0)
    plsc.subcore_barrier()

    def rdout(t, carry):
        row0 = s * ZR + t * 32
        pltpu.sync_copy(acc_sh.at[pl.ds(row0, 32)], rd_v)

        def pack(r, carry2):
            pk_v[r, :] = rd_v[r, pl.ds(0, 16)]
            return carry2

        lax.fori_loop(0, 32, pack, 0)
        pltpu.sync_copy(pk_v, den_out.at[c, pl.ds(row0, 32)])
        return carry

    lax.fori_loop(0, ZR // 32, rdout, 0)


BN = 1000


def _p2_body(x_ref, sc_ref, wl_ref, wr_ref, bl_ref, br_ref, we_ref,
             aatt_ref, g1_ref, b1_ref,
             xl_ref, xlb_ref, xrb_ref, exl_ref):
    x = x_ref[...]
    mu = jnp.mean(x, axis=-1, keepdims=True)
    xc = x - mu
    var = jnp.mean(xc * xc, axis=-1, keepdims=True)
    ln1 = xc / jnp.sqrt(var + 1e-5) * g1_ref[...] + b1_ref[...]
    xl = jnp.dot(ln1, wl_ref[...], preferred_element_type=jnp.float32) + bl_ref[...]
    xr = jnp.dot(ln1, wr_ref[...], preferred_element_type=jnp.float32) + br_ref[...]
    xl_ref[...] = xl
    xlb_ref[...] = xl.astype(jnp.bfloat16)
    xrb_ref[...] = xr.astype(jnp.bfloat16)
    ssum = sc_ref[0][:, :DE] + sc_ref[1][:, :DE]
    cnt = sc_ref[0][:, DE:DE + 1] + sc_ref[1][:, DE:DE + 1]
    la = ssum / jnp.maximum(cnt, 1.0)
    lee = jnp.dot(la, we_ref[...], preferred_element_type=jnp.float32)
    ml = xl + xr + lee
    ml = jnp.where(ml > 0, ml, 0.2 * ml)
    al = jnp.dot(ml, aatt_ref[...], preferred_element_type=jnp.float32)
    exl = jnp.exp(al)
    exl_ref[...] = jnp.concatenate([exl, jnp.zeros_like(exl)], axis=1)


def _p2_call(x, sc, WlT, WrT, bl2, br2, WeT, A_att, g1, b1):
    nb = N // BN
    full = lambda i: (0, 0)
    return pl.pallas_call(
        _p2_body,
        grid=(nb,),
        in_specs=[
            pl.BlockSpec((BN, D), lambda i: (i, 0)),
            pl.BlockSpec((NC, BN, 2 * DE), lambda i: (0, i, 0)),
            pl.BlockSpec((D, D), full),
            pl.BlockSpec((D, D), full),
            pl.BlockSpec((1, D), full),
            pl.BlockSpec((1, D), full),
            pl.BlockSpec((DE, D), full),
            pl.BlockSpec((D, H), full),
            pl.BlockSpec((1, D), full),
            pl.BlockSpec((1, D), full),
        ],
        out_specs=[
            pl.BlockSpec((BN, D), lambda i: (i, 0)),
            pl.BlockSpec((BN, D), lambda i: (i, 0)),
            pl.BlockSpec((BN, D), lambda i: (i, 0)),
            pl.BlockSpec((BN, DE), lambda i: (i, 0)),
        ],
        out_shape=[
            jax.ShapeDtypeStruct((N, D), jnp.float32),
            jax.ShapeDtypeStruct((N, D), jnp.bfloat16),
            jax.ShapeDtypeStruct((N, D), jnp.bfloat16),
            jax.ShapeDtypeStruct((N, DE), jnp.float32),
        ],
    )(x, sc, WlT, WrT, bl2, br2, WeT, A_att, g1, b1)


BE = 2048


def _p4_body(ea_ref, gl_ref, gr_ref, we_ref, aatt_ref, e8_ref,
             y_ref, exz_ref):
    gl = gl_ref[...].astype(jnp.float32)
    gr = gr_ref[...].astype(jnp.float32)
    ee = jnp.dot(ea_ref[...], we_ref[...], preferred_element_type=jnp.float32)
    m = gl + gr + ee
    m = jnp.where(m > 0, m, 0.2 * m)
    a = jnp.dot(m, aatt_ref[...], preferred_element_type=jnp.float32)
    ex = jnp.exp(a)
    exz_ref[...] = jnp.concatenate([ex, jnp.zeros_like(ex)], axis=1)
    y = jnp.dot(ex, e8_ref[...], preferred_element_type=jnp.float32) * gl
    y_ref[...] = jnp.stack([y[:, : D // 2], y[:, D // 2 :]])


def _p4_call(ea, gl, gr, WeT, A_att, E8):
    nb = EP // BE
    full = lambda i: (0, 0)
    return pl.pallas_call(
        _p4_body,
        grid=(nb,),
        in_specs=[
            pl.BlockSpec((BE, DE), lambda i: (i, 0)),
            pl.BlockSpec((BE, D), lambda i: (i, 0)),
            pl.BlockSpec((BE, D), lambda i: (i, 0)),
            pl.BlockSpec((DE, D), full),
            pl.BlockSpec((D, H), full),
            pl.BlockSpec((H, D), full),
        ],
        out_specs=[
            pl.BlockSpec((NC, BE, D // 2), lambda i: (0, i, 0)),
            pl.BlockSpec((BE, DE), lambda i: (i, 0)),
        ],
        out_shape=[
            jax.ShapeDtypeStruct((NC, EP, D // 2), jnp.float32),
            jax.ShapeDtypeStruct((EP, DE), jnp.float32),
        ],
    )(ea, gl, gr, WeT, A_att, E8)


def _p6_body(x_ref, xl_ref, exl_ref, num_ref, den_ref, e8_ref, gb_ref,
             g2_ref, b2g_ref, w1_ref, b1f_ref, w2_ref, b2f_ref,
             out_ref):
    x = x_ref[...]
    xl = xl_ref[...]
    exl = exl_ref[...][:, :H]
    num = jnp.concatenate([num_ref[0], num_ref[1]], axis=1)
    e8 = e8_ref[...]
    num = num + jnp.dot(exl, e8, preferred_element_type=jnp.float32) * xl
    den = den_ref[0][:, :H] + den_ref[1][:, :H] + exl
    den256 = jnp.dot(den, e8, preferred_element_type=jnp.float32)
    sa = num / den256 + gb_ref[...]
    x1 = x + sa
    mu = jnp.mean(x1, axis=-1, keepdims=True)
    xc = x1 - mu
    var = jnp.mean(xc * xc, axis=-1, keepdims=True)
    h = xc / jnp.sqrt(var + 1e-5) * g2_ref[...] + b2g_ref[...]
    f = jnp.dot(h, w1_ref[...], preferred_element_type=jnp.float32) + b1f_ref[...]
    f = 0.5 * f * (1.0 + lax.erf(f * 0.7071067811865476))
    ff = jnp.dot(f, w2_ref[...], preferred_element_type=jnp.float32) + b2f_ref[...]
    out_ref[...] = x1 + ff


def _p6_call(x, xl, exl, num, den, E8, gb, g2, b2g, W1T, b1f, W2T, b2f):
    nb = N // BN
    full = lambda i: (0, 0)
    return pl.pallas_call(
        _p6_body,
        grid=(nb,),
        in_specs=[
            pl.BlockSpec((BN, D), lambda i: (i, 0)),
            pl.BlockSpec((BN, D), lambda i: (i, 0)),
            pl.BlockSpec((BN, DE), lambda i: (i, 0)),
            pl.BlockSpec((NC, BN, D // 2), lambda i: (0, i, 0)),
            pl.BlockSpec((NC, BN, DE), lambda i: (0, i, 0)),
            pl.BlockSpec((H, D), full),
            pl.BlockSpec((1, D), full),
            pl.BlockSpec((1, D), full),
            pl.BlockSpec((1, D), full),
            pl.BlockSpec((D, 2 * D), full),
            pl.BlockSpec((1, 2 * D), full),
            pl.BlockSpec((2 * D, D), full),
            pl.BlockSpec((1, D), full),
        ],
        out_specs=pl.BlockSpec((BN, D), lambda i: (i, 0)),
        out_shape=jax.ShapeDtypeStruct((N, D), jnp.float32),
    )(x, xl, exl, num, den, E8, gb, g2, b2g, W1T, b1f, W2T, b2f)


def kernel(x, edge_index, edge_attr, Wl, bl, Wr, br, We, att, gat_bias,
           ln1_g, ln1_b, ln2_g, ln2_b, W1, b1, W2, b2):
    pad = EP - E
    src = jnp.concatenate([edge_index[0], jnp.zeros((pad,), edge_index.dtype)])
    dst_g = jnp.concatenate([edge_index[1], jnp.zeros((pad,), edge_index.dtype)])
    dst = jnp.concatenate([edge_index[1], jnp.full((pad,), TRASH, edge_index.dtype)])
    ea_p = jnp.concatenate([edge_attr, jnp.zeros((pad, DE), edge_attr.dtype)])
    WlT = Wl.T
    WrT = Wr.T
    WeT = We.T
    W1T = W1.T
    W2T = W2.T
    A_att = (jnp.zeros((D, H), jnp.float32)
             .at[jnp.arange(D), jnp.arange(D) // C].set(att.reshape(-1)))
    E8 = (jnp.arange(D)[None, :] // C == jnp.arange(H)[:, None]).astype(jnp.float32)
    z128 = jnp.zeros((NPAD, D // 2), jnp.float32)
    r2 = lambda v: v.reshape(1, -1)

    sc = _p1_attr_sums(dst, ea_p, z128)
    xl, xlb, xrb, exl = _p2_call(x, sc, WlT, WrT, r2(bl), r2(br), WeT,
                                 A_att, r2(ln1_g), r2(ln1_b))
    as_i32 = lambda a: jax.lax.bitcast_convert_type(a.reshape(-1, D // 2, 2), jnp.int32)
    as_bf16 = lambda a: jax.lax.bitcast_convert_type(a, jnp.bfloat16).reshape(EP, D)
    gl, gr = _p3_gather(src, dst_g, as_i32(xlb), as_i32(xrb))
    y, exz = _p4_call(ea_p, as_bf16(gl.reshape(EP, D // 2)),
                      as_bf16(gr.reshape(EP, D // 2)), WeT, A_att, E8)
    num = _p5_scatter(dst.reshape(NS, ITER, CB), y, z128)
    den = _p5b_den_scatter(dst, exz, z128)
    out = _p6_call(x, xl, exl, num, den, E8, r2(gat_bias), r2(ln2_g), r2(ln2_b),
                   W1T, r2(b1), W2T, r2(b2))
    return out


# speedup vs baseline: 2.4781x; 2.4781x over previous
import functools

import jax
import jax.numpy as jnp
from jax import lax
from jax.experimental import pallas as pl
from jax.experimental.pallas import tpu as pltpu
from jax.experimental.pallas import tpu_sc as plsc

N = 10000
E = 160000
D = 256
DE = 16
H = 8
C = 32

NC = 2
NS = 16
NW = NC * NS

NPAD = 10240
ZR = NPAD // NS
TRASH = N

CB = 128
EP = 163840
EPT = EP // NW
EPS = EP // NS

_mesh = plsc.VectorSubcoreMesh(core_axis_name="c", subcore_axis_name="s",
                               num_cores=NC, num_subcores=NS)


@functools.partial(
    pl.kernel,
    out_type=jax.ShapeDtypeStruct((NC, NPAD, 2 * DE), jnp.float32),
    mesh=_mesh,
    scratch_types=[
        pltpu.VMEM((CB,), jnp.int32),
        pltpu.VMEM((CB, DE), jnp.float32),
        pltpu.VMEM((CB, 128), jnp.float32),
        pltpu.VMEM((32, 128), jnp.float32),
        pltpu.VMEM((32, 2 * DE), jnp.float32),
        pltpu.VMEM_SHARED((NPAD, 128), jnp.float32),
    ],
)
def _p1_attr_sums(dst_hbm, ea_hbm, z128_hbm,
                  sc_out,
                  idx_v, ea_v, wide_v, rd_v, pk_v, acc_sh):
    c = lax.axis_index("c")
    s = lax.axis_index("s")
    wid = s * NC + c
    pltpu.sync_copy(z128_hbm.at[pl.ds(s * ZR, ZR)], acc_sh.at[pl.ds(s * ZR, ZR)])
    zv = jnp.zeros((16,), jnp.float32)
    onev = jnp.where(lax.iota(jnp.int32, 16) == 0, 1.0, 0.0)

    def zrow(r, carry):
        for k in range(8):
            wide_v[r, pl.ds(16 * k, 16)] = zv
        return carry

    lax.fori_loop(0, CB, zrow, 0)
    plsc.subcore_barrier()

    def body(i, carry):
        base = wid * EPT + i * CB
        pltpu.sync_copy(dst_hbm.at[pl.ds(base, CB)], idx_v)
        pltpu.sync_copy(ea_hbm.at[pl.ds(base, CB)], ea_v)

        def fill(r, carry2):
            wide_v[r, pl.ds(0, 16)] = ea_v[r, :]
            wide_v[r, pl.ds(16, 16)] = onev
            return carry2

        lax.fori_loop(0, CB, fill, 0)
        pltpu.sync_copy(wide_v, acc_sh.at[idx_v], add=True)
        return carry

    lax.fori_loop(0, EPT // CB, body, 0)
    plsc.subcore_barrier()

    def rdout(t, carry):
        row0 = s * ZR + t * 32
        pltpu.sync_copy(acc_sh.at[pl.ds(row0, 32)], rd_v)

        def pack(r, carry2):
            pk_v[r, pl.ds(0, 16)] = rd_v[r, pl.ds(0, 16)]
            pk_v[r, pl.ds(16, 16)] = rd_v[r, pl.ds(16, 16)]
            return carry2

        lax.fori_loop(0, 32, pack, 0)
        pltpu.sync_copy(pk_v, sc_out.at[c, pl.ds(row0, 32)])
        return carry

    lax.fori_loop(0, ZR // 32, rdout, 0)


ITER = EPS // CB
SG = 2
ITER2 = EPS // (SG * CB)
NCHUNK = EP // CB


def _p3_one_stream(idx_hbm, table_hbm, out_hbm, s, idx_v, buf, g0, g1, w0, w1):
    base0 = s * EPS
    pltpu.sync_copy(idx_hbm.at[pl.ds(base0, EPS)], idx_v)
    b0 = buf.at[0]
    b1 = buf.at[1]

    def gather(i, b, sem):
        for k in range(SG):
            pltpu.async_copy(
                table_hbm.at[idx_v.at[pl.ds((i * SG + k) * CB, CB)]],
                b.at[k], sem)

    def wb(i, b, sem):
        return pltpu.async_copy(b, out_hbm.at[pl.ds(s * ITER2 * SG + i * SG, SG)], sem)

    def wait_g(b, sem):
        pltpu.make_async_copy(out_hbm.at[pl.ds(0, SG)], b, sem).wait()

    def wait_w(b, sem):
        pltpu.make_async_copy(b, out_hbm.at[pl.ds(0, SG)], sem).wait()

    gather(0, b0, g0)
    J = ITER2 // 2

    def body(j, carry):
        i = 2 * j
        wait_g(b0, g0)
        wb(i, b0, w0)

        @pl.when(j > 0)
        def _():
            wait_w(b1, w1)

        gather(i + 1, b1, g1)
        wait_g(b1, g1)
        wb(i + 1, b1, w1)

        @pl.when(j < J - 1)
        def _():
            wait_w(b0, w0)
            gather(i + 2, b0, g0)

        return carry

    lax.fori_loop(0, J, body, 0)
    wait_w(b0, w0)
    wait_w(b1, w1)


@functools.partial(
    pl.kernel,
    out_type=[
        jax.ShapeDtypeStruct((NCHUNK, CB, D // 2), jnp.int32),
        jax.ShapeDtypeStruct((NCHUNK, CB, D // 2), jnp.int32),
    ],
    mesh=_mesh,
    scratch_types=[
        pltpu.VMEM((EPS,), jnp.int32),
        pltpu.VMEM((2, SG, CB, D // 2), jnp.int32),
        pltpu.SemaphoreType.DMA,
        pltpu.SemaphoreType.DMA,
        pltpu.SemaphoreType.DMA,
        pltpu.SemaphoreType.DMA,
    ],
)
def _p3_gather(src_hbm, dst_hbm, xlb_hbm, xrb_hbm,
               gl_out, gr_out,
               idx_v, buf, g0, g1, w0, w1):
    c = lax.axis_index("c")
    s = lax.axis_index("s")

    @pl.when(c == 0)
    def _():
        _p3_one_stream(src_hbm, xlb_hbm, gl_out, s, idx_v, buf, g0, g1, w0, w1)

    @pl.when(c == 1)
    def _():
        _p3_one_stream(dst_hbm, xrb_hbm, gr_out, s, idx_v, buf, g0, g1, w0, w1)


NP5 = 10112
ZR5 = NP5 // NS


@functools.partial(
    pl.kernel,
    out_type=jax.ShapeDtypeStruct((NC, NPAD, D // 2), jnp.float32),
    mesh=_mesh,
    scratch_types=[
        pltpu.VMEM((ITER, CB), jnp.int32),
        pltpu.VMEM((2, CB, D // 2), jnp.float32),
        pltpu.VMEM_SHARED((NP5, D // 2), jnp.float32),
        pltpu.SemaphoreType.DMA,
        pltpu.SemaphoreType.DMA,
        pltpu.SemaphoreType.DMA,
        pltpu.SemaphoreType.DMA,
    ],
)
def _p5_scatter(dst_r_hbm, y_hbm, z128_hbm,
                num_out,
                idx_v, buf, acc_sh, l0, l1, a0, a1):
    c = lax.axis_index("c")
    s = lax.axis_index("s")
    pltpu.sync_copy(z128_hbm.at[pl.ds(s * ZR5, ZR5)], acc_sh.at[pl.ds(s * ZR5, ZR5)])
    pltpu.sync_copy(dst_r_hbm.at[s], idx_v)
    plsc.subcore_barrier()
    b0 = buf.at[0]
    b1 = buf.at[1]
    base0 = s * EPS

    def load(i, b, sem):
        return pltpu.async_copy(y_hbm.at[c, pl.ds(base0 + i * CB, CB)], b, sem)

    def scat(i, b, sem):
        return pltpu.async_copy(b, acc_sh.at[idx_v.at[i]], sem, add=True)

    def wait_l(b, sem):
        pltpu.make_async_copy(y_hbm.at[c, pl.ds(0, CB)], b, sem).wait()

    def wait_a(b, sem):
        pltpu.make_async_copy(b, acc_sh.at[pl.ds(0, CB)], sem).wait()

    load(0, b0, l0)
    J = ITER // 2

    def body(j, carry):
        i = 2 * j
        wait_l(b0, l0)
        scat(i, b0, a0)

        @pl.when(j > 0)
        def _():
            wait_a(b1, a1)

        load(i + 1, b1, l1)
        wait_l(b1, l1)
        scat(i + 1, b1, a1)

        @pl.when(j < J - 1)
        def _():
            wait_a(b0, a0)
            load(i + 2, b0, l0)

        return carry

    lax.fori_loop(0, J, body, 0)
    wait_a(b0, a0)
    wait_a(b1, a1)
    plsc.subcore_barrier()
    pltpu.sync_copy(acc_sh.at[pl.ds(s * ZR5, ZR5)], num_out.at[c, pl.ds(s * ZR5, ZR5)])


@functools.partial(
    pl.kernel,
    out_type=jax.ShapeDtypeStruct((NC, NPAD, DE), jnp.float32),
    mesh=_mesh,
    scratch_types=[
        pltpu.VMEM((CB,), jnp.int32),
        pltpu.VMEM((CB, DE), jnp.float32),
        pltpu.VMEM((CB, 128), jnp.float32),
        pltpu.VMEM((32, 128), jnp.float32),
        pltpu.VMEM((32, DE), jnp.float32),
        pltpu.VMEM_SHARED((NPAD, 128), jnp.float32),
    ],
)
def _p5b_den_scatter(dst_hbm, exz_hbm, z128_hbm,
                     den_out,
                     idx_v, ex_v, wide_v, rd_v, pk_v, acc_sh):
    c = lax.axis_index("c")
    s = lax.axis_index("s")
    wid = s * NC + c
    pltpu.sync_copy(z128_hbm.at[pl.ds(s * ZR, ZR)], acc_sh.at[pl.ds(s * ZR, ZR)])
    zv = jnp.zeros((16,), jnp.float32)

    def zrow(r, carry):
        for k in range(8):
            wide_v[r, pl.ds(16 * k, 16)] = zv
        return carry

    lax.fori_loop(0, CB, zrow, 0)
    plsc.subcore_barrier()

    def body(i, carry):
        base = wid * EPT + i * CB
        pltpu.sync_copy(dst_hbm.at[pl.ds(base, CB)], idx_v)
        pltpu.sync_copy(exz_hbm.at[pl.ds(base, CB)], ex_v)

        def fill(r, carry2):
            wide_v[r, pl.ds(0, 16)] = ex_v[r, :]
            return carry2

        lax.fori_loop(0, CB, fill, 0)
        pltpu.sync_copy(wide_v, acc_sh.at[idx_v], add=True)
        return carry

    lax.fori_loop(0, EPT // CB, body, 0)
    plsc.subcore_barrier()

    def rdout(t, carry):
        row0 = s * ZR + t * 32
        pltpu.sync_copy(acc_sh.at[pl.ds(row0, 32)], rd_v)

        def pack(r, carry2):
            pk_v[r, :] = rd_v[r, pl.ds(0, 16)]
            return carry2

        lax.fori_loop(0, 32, pack, 0)
        pltpu.sync_copy(pk_v, den_out.at[c, pl.ds(row0, 32)])
        return carry

    lax.fori_loop(0, ZR // 32, rdout, 0)


BN = 1000


def _pack2(v):
    lo = jax.lax.bitcast_convert_type(
        v[:, :D // 2].astype(jnp.bfloat16).astype(jnp.float32), jnp.int32)
    hi = jax.lax.bitcast_convert_type(
        v[:, D // 2:].astype(jnp.bfloat16).astype(jnp.float32), jnp.int32)
    return jax.lax.shift_right_logical(lo, 16) | (hi & jnp.int32(-65536))


def _unpack2(w):
    lo = jax.lax.bitcast_convert_type(jax.lax.shift_left(w, 16), jnp.float32)
    hi = jax.lax.bitcast_convert_type(w & jnp.int32(-65536), jnp.float32)
    return jnp.concatenate([lo, hi], axis=1)


def _p2_body(x_ref, sc_ref, wl_ref, wr_ref, bl_ref, br_ref, we_ref,
             aatt_ref, g1_ref, b1_ref,
             xl_ref, xlb_ref, xrb_ref, exl_ref):
    x = x_ref[...]
    mu = jnp.mean(x, axis=-1, keepdims=True)
    xc = x - mu
    var = jnp.mean(xc * xc, axis=-1, keepdims=True)
    ln1 = xc / jnp.sqrt(var + 1e-5) * g1_ref[...] + b1_ref[...]
    xl = jnp.dot(ln1, wl_ref[...], preferred_element_type=jnp.float32) + bl_ref[...]
    xr = jnp.dot(ln1, wr_ref[...], preferred_element_type=jnp.float32) + br_ref[...]
    xl_ref[...] = xl
    xlb_ref[...] = _pack2(xl)
    xrb_ref[...] = _pack2(xr)
    ssum = sc_ref[0][:, :DE] + sc_ref[1][:, :DE]
    cnt = sc_ref[0][:, DE:DE + 1] + sc_ref[1][:, DE:DE + 1]
    la = ssum / jnp.maximum(cnt, 1.0)
    lee = jnp.dot(la, we_ref[...], preferred_element_type=jnp.float32)
    ml = xl + xr + lee
    ml = jnp.where(ml > 0, ml, 0.2 * ml)
    al = jnp.dot(ml, aatt_ref[...], preferred_element_type=jnp.float32)
    exl = jnp.exp(al)
    exl_ref[...] = jnp.concatenate([exl, jnp.zeros_like(exl)], axis=1)


def _p2_call(x, sc, WlT, WrT, bl2, br2, WeT, A_att, g1, b1):
    nb = N // BN
    full = lambda i: (0, 0)
    return pl.pallas_call(
        _p2_body,
        grid=(nb,),
        in_specs=[
            pl.BlockSpec((BN, D), lambda i: (i, 0)),
            pl.BlockSpec((NC, BN, 2 * DE), lambda i: (0, i, 0)),
            pl.BlockSpec((D, D), full),
            pl.BlockSpec((D, D), full),
            pl.BlockSpec((1, D), full),
            pl.BlockSpec((1, D), full),
            pl.BlockSpec((DE, D), full),
            pl.BlockSpec((D, H), full),
            pl.BlockSpec((1, D), full),
            pl.BlockSpec((1, D), full),
        ],
        out_specs=[
            pl.BlockSpec((BN, D), lambda i: (i, 0)),
            pl.BlockSpec((BN, D // 2), lambda i: (i, 0)),
            pl.BlockSpec((BN, D // 2), lambda i: (i, 0)),
            pl.BlockSpec((BN, DE), lambda i: (i, 0)),
        ],
        out_shape=[
            jax.ShapeDtypeStruct((N, D), jnp.float32),
            jax.ShapeDtypeStruct((N, D // 2), jnp.int32),
            jax.ShapeDtypeStruct((N, D // 2), jnp.int32),
            jax.ShapeDtypeStruct((N, DE), jnp.float32),
        ],
    )(x, sc, WlT, WrT, bl2, br2, WeT, A_att, g1, b1)


BE = 2048


def _p4_body(ea_ref, gl_ref, gr_ref, we_ref, aatt_ref, e8_ref,
             y_ref, exz_ref):
    gl = _unpack2(gl_ref[...])
    gr = _unpack2(gr_ref[...])
    ee = jnp.dot(ea_ref[...], we_ref[...], preferred_element_type=jnp.float32)
    m = gl + gr + ee
    m = jnp.where(m > 0, m, 0.2 * m)
    a = jnp.dot(m, aatt_ref[...], preferred_element_type=jnp.float32)
    ex = jnp.exp(a)
    exz_ref[...] = jnp.concatenate([ex, jnp.zeros_like(ex)], axis=1)
    y = jnp.dot(ex, e8_ref[...], preferred_element_type=jnp.float32) * gl
    y_ref[...] = jnp.stack([y[:, : D // 2], y[:, D // 2 :]])


def _p4_call(ea, gl, gr, WeT, A_att, E8):
    nb = EP // BE
    full = lambda i: (0, 0)
    return pl.pallas_call(
        _p4_body,
        grid=(nb,),
        in_specs=[
            pl.BlockSpec((BE, DE), lambda i: (i, 0)),
            pl.BlockSpec((BE, D // 2), lambda i: (i, 0)),
            pl.BlockSpec((BE, D // 2), lambda i: (i, 0)),
            pl.BlockSpec((DE, D), full),
            pl.BlockSpec((D, H), full),
            pl.BlockSpec((H, D), full),
        ],
        out_specs=[
            pl.BlockSpec((NC, BE, D // 2), lambda i: (0, i, 0)),
            pl.BlockSpec((BE, DE), lambda i: (i, 0)),
        ],
        out_shape=[
            jax.ShapeDtypeStruct((NC, EP, D // 2), jnp.float32),
            jax.ShapeDtypeStruct((EP, DE), jnp.float32),
        ],
    )(ea, gl, gr, WeT, A_att, E8)


def _p6_body(x_ref, xl_ref, exl_ref, num_ref, den_ref, e8_ref, gb_ref,
             g2_ref, b2g_ref, w1_ref, b1f_ref, w2_ref, b2f_ref,
             out_ref):
    x = x_ref[...]
    xl = xl_ref[...]
    exl = exl_ref[...][:, :H]
    num = jnp.concatenate([num_ref[0], num_ref[1]], axis=1)
    e8 = e8_ref[...]
    num = num + jnp.dot(exl, e8, preferred_element_type=jnp.float32) * xl
    den = den_ref[0][:, :H] + den_ref[1][:, :H] + exl
    den256 = jnp.dot(den, e8, preferred_element_type=jnp.float32)
    sa = num / den256 + gb_ref[...]
    x1 = x + sa
    mu = jnp.mean(x1, axis=-1, keepdims=True)
    xc = x1 - mu
    var = jnp.mean(xc * xc, axis=-1, keepdims=True)
    h = xc / jnp.sqrt(var + 1e-5) * g2_ref[...] + b2g_ref[...]
    f = jnp.dot(h, w1_ref[...], preferred_element_type=jnp.float32) + b1f_ref[...]
    f = 0.5 * f * (1.0 + lax.erf(f * 0.7071067811865476))
    ff = jnp.dot(f, w2_ref[...], preferred_element_type=jnp.float32) + b2f_ref[...]
    out_ref[...] = x1 + ff


def _p6_call(x, xl, exl, num, den, E8, gb, g2, b2g, W1T, b1f, W2T, b2f):
    nb = N // BN
    full = lambda i: (0, 0)
    return pl.pallas_call(
        _p6_body,
        grid=(nb,),
        in_specs=[
            pl.BlockSpec((BN, D), lambda i: (i, 0)),
            pl.BlockSpec((BN, D), lambda i: (i, 0)),
            pl.BlockSpec((BN, DE), lambda i: (i, 0)),
            pl.BlockSpec((NC, BN, D // 2), lambda i: (0, i, 0)),
            pl.BlockSpec((NC, BN, DE), lambda i: (0, i, 0)),
            pl.BlockSpec((H, D), full),
            pl.BlockSpec((1, D), full),
            pl.BlockSpec((1, D), full),
            pl.BlockSpec((1, D), full),
            pl.BlockSpec((D, 2 * D), full),
            pl.BlockSpec((1, 2 * D), full),
            pl.BlockSpec((2 * D, D), full),
            pl.BlockSpec((1, D), full),
        ],
        out_specs=pl.BlockSpec((BN, D), lambda i: (i, 0)),
        out_shape=jax.ShapeDtypeStruct((N, D), jnp.float32),
    )(x, xl, exl, num, den, E8, gb, g2, b2g, W1T, b1f, W2T, b2f)


def kernel(x, edge_index, edge_attr, Wl, bl, Wr, br, We, att, gat_bias,
           ln1_g, ln1_b, ln2_g, ln2_b, W1, b1, W2, b2):
    pad = EP - E
    src = jnp.concatenate([edge_index[0], jnp.zeros((pad,), edge_index.dtype)])
    dst_g = jnp.concatenate([edge_index[1], jnp.zeros((pad,), edge_index.dtype)])
    dst = jnp.concatenate([edge_index[1], jnp.full((pad,), TRASH, edge_index.dtype)])
    ea_p = jnp.concatenate([edge_attr, jnp.zeros((pad, DE), edge_attr.dtype)])
    WlT = Wl.T
    WrT = Wr.T
    WeT = We.T
    W1T = W1.T
    W2T = W2.T
    A_att = (jnp.zeros((D, H), jnp.float32)
             .at[jnp.arange(D), jnp.arange(D) // C].set(att.reshape(-1)))
    E8 = (jnp.arange(D)[None, :] // C == jnp.arange(H)[:, None]).astype(jnp.float32)
    z128 = jnp.zeros((NPAD, D // 2), jnp.float32)
    r2 = lambda v: v.reshape(1, -1)

    sc = _p1_attr_sums(dst, ea_p, z128)
    xl, xlp, xrp, exl = _p2_call(x, sc, WlT, WrT, r2(bl), r2(br), WeT,
                                 A_att, r2(ln1_g), r2(ln1_b))
    gl, gr = _p3_gather(src, dst_g, xlp, xrp)
    y, exz = _p4_call(ea_p, gl.reshape(EP, D // 2), gr.reshape(EP, D // 2),
                      WeT, A_att, E8)
    num = _p5_scatter(dst.reshape(NS, ITER, CB), y, z128)
    den = _p5b_den_scatter(dst, exz, z128)
    out = _p6_call(x, xl, exl, num, den, E8, r2(gat_bias), r2(ln2_g), r2(ln2_b),
                   W1T, r2(b1), W2T, r2(b2))
    return out


# speedup vs baseline: 2.6720x; 1.0783x over previous
import functools

import jax
import jax.numpy as jnp
from jax import lax
from jax.experimental import pallas as pl
from jax.experimental.pallas import tpu as pltpu
from jax.experimental.pallas import tpu_sc as plsc

N = 10000
E = 160000
D = 256
DE = 16
H = 8
C = 32

NC = 2
NS = 16
NW = NC * NS

NPAD = 10240
ZR = NPAD // NS
TRASH = N

CB = 128
CBN = 64
EP = 163840
EPT = EP // NW
EPS = EP // NS

_mesh = plsc.VectorSubcoreMesh(core_axis_name="c", subcore_axis_name="s",
                               num_cores=NC, num_subcores=NS)


@functools.partial(
    pl.kernel,
    out_type=jax.ShapeDtypeStruct((NC, NPAD, 2 * DE), jnp.float32),
    mesh=_mesh,
    scratch_types=[
        pltpu.VMEM((2, CBN), jnp.int32),
        pltpu.VMEM((2, CBN, DE), jnp.float32),
        pltpu.VMEM((CBN, 128), jnp.float32),
        pltpu.VMEM((32, 128), jnp.float32),
        pltpu.VMEM((32, 2 * DE), jnp.float32),
        pltpu.VMEM_SHARED((NPAD, 128), jnp.float32),
        pltpu.SemaphoreType.DMA,
        pltpu.SemaphoreType.DMA,
    ],
)
def _p1_attr_sums(dst_hbm, ea_hbm, z128_hbm,
                  sc_out,
                  idx_v, ea_v, wide_v, rd_v, pk_v, acc_sh, la0, la1):
    c = lax.axis_index("c")
    s = lax.axis_index("s")
    wid = s * NC + c
    pltpu.sync_copy(z128_hbm.at[pl.ds(s * ZR, ZR)], acc_sh.at[pl.ds(s * ZR, ZR)])
    zv = jnp.zeros((16,), jnp.float32)
    onev = jnp.where(lax.iota(jnp.int32, 16) == 0, 1.0, 0.0)

    def zrow(r, carry):
        for k in range(8):
            wide_v[r, pl.ds(16 * k, 16)] = zv
        return carry

    lax.fori_loop(0, CBN, zrow, 0)
    plsc.subcore_barrier()

    def load(i, b, sem):
        base = wid * EPT + i * CBN
        pltpu.async_copy(dst_hbm.at[pl.ds(base, CBN)], idx_v.at[b], sem)
        pltpu.async_copy(ea_hbm.at[pl.ds(base, CBN)], ea_v.at[b], sem)

    def wait_load(b, sem):
        pltpu.make_async_copy(dst_hbm.at[pl.ds(0, CBN)], idx_v.at[b], sem).wait()
        pltpu.make_async_copy(ea_hbm.at[pl.ds(0, CBN)], ea_v.at[b], sem).wait()

    def consume(b):
        def fill(r, carry2):
            wide_v[r, pl.ds(0, 16)] = ea_v[b, r, :]
            wide_v[r, pl.ds(16, 16)] = onev
            return carry2

        lax.fori_loop(0, CBN, fill, 0)
        pltpu.sync_copy(wide_v, acc_sh.at[idx_v.at[b]], add=True)

    load(0, 0, la0)
    J = (EPT // CBN) // 2

    def body(j, carry):
        i = 2 * j
        wait_load(0, la0)
        load(i + 1, 1, la1)
        consume(0)
        wait_load(1, la1)

        @pl.when(j < J - 1)
        def _():
            load(i + 2, 0, la0)

        consume(1)
        return carry

    lax.fori_loop(0, J, body, 0)
    plsc.subcore_barrier()

    def rdout(t, carry):
        row0 = s * ZR + t * 32
        pltpu.sync_copy(acc_sh.at[pl.ds(row0, 32)], rd_v)

        def pack(r, carry2):
            pk_v[r, pl.ds(0, 16)] = rd_v[r, pl.ds(0, 16)]
            pk_v[r, pl.ds(16, 16)] = rd_v[r, pl.ds(16, 16)]
            return carry2

        lax.fori_loop(0, 32, pack, 0)
        pltpu.sync_copy(pk_v, sc_out.at[c, pl.ds(row0, 32)])
        return carry

    lax.fori_loop(0, ZR // 32, rdout, 0)


ITER = EPS // CB
SG = 2
ITER2 = EPS // (SG * CB)
NCHUNK = EP // CB


def _p3_one_stream(idx_hbm, table_hbm, out_hbm, s, idx_v, buf, g0, g1, w0, w1):
    base0 = s * EPS
    pltpu.sync_copy(idx_hbm.at[pl.ds(base0, EPS)], idx_v)
    b0 = buf.at[0]
    b1 = buf.at[1]

    def gather(i, b, sem):
        for k in range(SG):
            pltpu.async_copy(
                table_hbm.at[idx_v.at[pl.ds((i * SG + k) * CB, CB)]],
                b.at[k], sem)

    def wb(i, b, sem):
        return pltpu.async_copy(b, out_hbm.at[pl.ds(s * ITER2 * SG + i * SG, SG)], sem)

    def wait_g(b, sem):
        pltpu.make_async_copy(out_hbm.at[pl.ds(0, SG)], b, sem).wait()

    def wait_w(b, sem):
        pltpu.make_async_copy(b, out_hbm.at[pl.ds(0, SG)], sem).wait()

    gather(0, b0, g0)
    J = ITER2 // 2

    def body(j, carry):
        i = 2 * j
        wait_g(b0, g0)
        wb(i, b0, w0)

        @pl.when(j > 0)
        def _():
            wait_w(b1, w1)

        gather(i + 1, b1, g1)
        wait_g(b1, g1)
        wb(i + 1, b1, w1)

        @pl.when(j < J - 1)
        def _():
            wait_w(b0, w0)
            gather(i + 2, b0, g0)

        return carry

    lax.fori_loop(0, J, body, 0)
    wait_w(b0, w0)
    wait_w(b1, w1)


@functools.partial(
    pl.kernel,
    out_type=[
        jax.ShapeDtypeStruct((NCHUNK, CB, D // 2), jnp.int32),
        jax.ShapeDtypeStruct((NCHUNK, CB, D // 2), jnp.int32),
    ],
    mesh=_mesh,
    scratch_types=[
        pltpu.VMEM((EPS,), jnp.int32),
        pltpu.VMEM((2, SG, CB, D // 2), jnp.int32),
        pltpu.SemaphoreType.DMA,
        pltpu.SemaphoreType.DMA,
        pltpu.SemaphoreType.DMA,
        pltpu.SemaphoreType.DMA,
    ],
)
def _p3_gather(src_hbm, dst_hbm, xlb_hbm, xrb_hbm,
               gl_out, gr_out,
               idx_v, buf, g0, g1, w0, w1):
    c = lax.axis_index("c")
    s = lax.axis_index("s")

    @pl.when(c == 0)
    def _():
        _p3_one_stream(src_hbm, xlb_hbm, gl_out, s, idx_v, buf, g0, g1, w0, w1)

    @pl.when(c == 1)
    def _():
        _p3_one_stream(dst_hbm, xrb_hbm, gr_out, s, idx_v, buf, g0, g1, w0, w1)


NP5 = 10112
ZR5 = NP5 // NS


@functools.partial(
    pl.kernel,
    out_type=jax.ShapeDtypeStruct((NC, NPAD, D // 2), jnp.float32),
    mesh=_mesh,
    scratch_types=[
        pltpu.VMEM((ITER, CB), jnp.int32),
        pltpu.VMEM((2, CB, D // 2), jnp.float32),
        pltpu.VMEM_SHARED((NP5, D // 2), jnp.float32),
        pltpu.SemaphoreType.DMA,
        pltpu.SemaphoreType.DMA,
        pltpu.SemaphoreType.DMA,
        pltpu.SemaphoreType.DMA,
    ],
)
def _p5_scatter(dst_r_hbm, y_hbm, z128_hbm,
                num_out,
                idx_v, buf, acc_sh, l0, l1, a0, a1):
    c = lax.axis_index("c")
    s = lax.axis_index("s")
    pltpu.sync_copy(z128_hbm.at[pl.ds(s * ZR5, ZR5)], acc_sh.at[pl.ds(s * ZR5, ZR5)])
    pltpu.sync_copy(dst_r_hbm.at[s], idx_v)
    plsc.subcore_barrier()
    b0 = buf.at[0]
    b1 = buf.at[1]
    base0 = s * EPS

    def load(i, b, sem):
        return pltpu.async_copy(y_hbm.at[c, pl.ds(base0 + i * CB, CB)], b, sem)

    def scat(i, b, sem):
        return pltpu.async_copy(b, acc_sh.at[idx_v.at[i]], sem, add=True)

    def wait_l(b, sem):
        pltpu.make_async_copy(y_hbm.at[c, pl.ds(0, CB)], b, sem).wait()

    def wait_a(b, sem):
        pltpu.make_async_copy(b, acc_sh.at[pl.ds(0, CB)], sem).wait()

    load(0, b0, l0)
    J = ITER // 2

    def body(j, carry):
        i = 2 * j
        wait_l(b0, l0)
        scat(i, b0, a0)

        @pl.when(j > 0)
        def _():
            wait_a(b1, a1)

        load(i + 1, b1, l1)
        wait_l(b1, l1)
        scat(i + 1, b1, a1)

        @pl.when(j < J - 1)
        def _():
            wait_a(b0, a0)
            load(i + 2, b0, l0)

        return carry

    lax.fori_loop(0, J, body, 0)
    wait_a(b0, a0)
    wait_a(b1, a1)
    plsc.subcore_barrier()
    pltpu.sync_copy(acc_sh.at[pl.ds(s * ZR5, ZR5)], num_out.at[c, pl.ds(s * ZR5, ZR5)])


@functools.partial(
    pl.kernel,
    out_type=jax.ShapeDtypeStruct((NC, NPAD, DE), jnp.float32),
    mesh=_mesh,
    scratch_types=[
        pltpu.VMEM((2, CBN), jnp.int32),
        pltpu.VMEM((2, CBN, DE), jnp.float32),
        pltpu.VMEM((CBN, 128), jnp.float32),
        pltpu.VMEM((32, 128), jnp.float32),
        pltpu.VMEM((32, DE), jnp.float32),
        pltpu.VMEM_SHARED((NPAD, 128), jnp.float32),
        pltpu.SemaphoreType.DMA,
        pltpu.SemaphoreType.DMA,
    ],
)
def _p5b_den_scatter(dst_hbm, exz_hbm, z128_hbm,
                     den_out,
                     idx_v, ex_v, wide_v, rd_v, pk_v, acc_sh, la0, la1):
    c = lax.axis_index("c")
    s = lax.axis_index("s")
    wid = s * NC + c
    pltpu.sync_copy(z128_hbm.at[pl.ds(s * ZR, ZR)], acc_sh.at[pl.ds(s * ZR, ZR)])
    zv = jnp.zeros((16,), jnp.float32)

    def zrow(r, carry):
        for k in range(8):
            wide_v[r, pl.ds(16 * k, 16)] = zv
        return carry

    lax.fori_loop(0, CBN, zrow, 0)
    plsc.subcore_barrier()

    def load(i, b, sem):
        base = wid * EPT + i * CBN
        pltpu.async_copy(dst_hbm.at[pl.ds(base, CBN)], idx_v.at[b], sem)
        pltpu.async_copy(exz_hbm.at[pl.ds(base, CBN)], ex_v.at[b], sem)

    def wait_load(b, sem):
        pltpu.make_async_copy(dst_hbm.at[pl.ds(0, CBN)], idx_v.at[b], sem).wait()
        pltpu.make_async_copy(exz_hbm.at[pl.ds(0, CBN)], ex_v.at[b], sem).wait()

    def consume(b):
        def fill(r, carry2):
            wide_v[r, pl.ds(0, 16)] = ex_v[b, r, :]
            return carry2

        lax.fori_loop(0, CBN, fill, 0)
        pltpu.sync_copy(wide_v, acc_sh.at[idx_v.at[b]], add=True)

    load(0, 0, la0)
    J = (EPT // CBN) // 2

    def body(j, carry):
        i = 2 * j
        wait_load(0, la0)
        load(i + 1, 1, la1)
        consume(0)
        wait_load(1, la1)

        @pl.when(j < J - 1)
        def _():
            load(i + 2, 0, la0)

        consume(1)
        return carry

    lax.fori_loop(0, J, body, 0)
    plsc.subcore_barrier()

    def rdout(t, carry):
        row0 = s * ZR + t * 32
        pltpu.sync_copy(acc_sh.at[pl.ds(row0, 32)], rd_v)

        def pack(r, carry2):
            pk_v[r, :] = rd_v[r, pl.ds(0, 16)]
            return carry2

        lax.fori_loop(0, 32, pack, 0)
        pltpu.sync_copy(pk_v, den_out.at[c, pl.ds(row0, 32)])
        return carry

    lax.fori_loop(0, ZR // 32, rdout, 0)


BN = 1000


def _pack2(v):
    lo = jax.lax.bitcast_convert_type(
        v[:, :D // 2].astype(jnp.bfloat16).astype(jnp.float32), jnp.int32)
    hi = jax.lax.bitcast_convert_type(
        v[:, D // 2:].astype(jnp.bfloat16).astype(jnp.float32), jnp.int32)
    return jax.lax.shift_right_logical(lo, 16) | (hi & jnp.int32(-65536))


def _unpack2(w):
    lo = jax.lax.bitcast_convert_type(jax.lax.shift_left(w, 16), jnp.float32)
    hi = jax.lax.bitcast_convert_type(w & jnp.int32(-65536), jnp.float32)
    return jnp.concatenate([lo, hi], axis=1)


def _p2_body(x_ref, sc_ref, wl_ref, wr_ref, bl_ref, br_ref, we_ref,
             aatt_ref, g1_ref, b1_ref,
             xl_ref, xlb_ref, xrb_ref, exl_ref):
    x = x_ref[...]
    mu = jnp.mean(x, axis=-1, keepdims=True)
    xc = x - mu
    var = jnp.mean(xc * xc, axis=-1, keepdims=True)
    ln1 = xc / jnp.sqrt(var + 1e-5) * g1_ref[...] + b1_ref[...]
    xl = jnp.dot(ln1, wl_ref[...], preferred_element_type=jnp.float32) + bl_ref[...]
    xr = jnp.dot(ln1, wr_ref[...], preferred_element_type=jnp.float32) + br_ref[...]
    xl_ref[...] = xl
    xlb_ref[...] = _pack2(xl)
    xrb_ref[...] = _pack2(xr)
    ssum = sc_ref[0][:, :DE] + sc_ref[1][:, :DE]
    cnt = sc_ref[0][:, DE:DE + 1] + sc_ref[1][:, DE:DE + 1]
    la = ssum / jnp.maximum(cnt, 1.0)
    lee = jnp.dot(la, we_ref[...], preferred_element_type=jnp.float32)
    ml = xl + xr + lee
    ml = jnp.where(ml > 0, ml, 0.2 * ml)
    al = jnp.dot(ml, aatt_ref[...], preferred_element_type=jnp.float32)
    exl = jnp.exp(al)
    exl_ref[...] = jnp.concatenate([exl, jnp.zeros_like(exl)], axis=1)


def _p2_call(x, sc, WlT, WrT, bl2, br2, WeT, A_att, g1, b1):
    nb = N // BN
    full = lambda i: (0, 0)
    return pl.pallas_call(
        _p2_body,
        grid=(nb,),
        in_specs=[
            pl.BlockSpec((BN, D), lambda i: (i, 0)),
            pl.BlockSpec((NC, BN, 2 * DE), lambda i: (0, i, 0)),
            pl.BlockSpec((D, D), full),
            pl.BlockSpec((D, D), full),
            pl.BlockSpec((1, D), full),
            pl.BlockSpec((1, D), full),
            pl.BlockSpec((DE, D), full),
            pl.BlockSpec((D, H), full),
            pl.BlockSpec((1, D), full),
            pl.BlockSpec((1, D), full),
        ],
        out_specs=[
            pl.BlockSpec((BN, D), lambda i: (i, 0)),
            pl.BlockSpec((BN, D // 2), lambda i: (i, 0)),
            pl.BlockSpec((BN, D // 2), lambda i: (i, 0)),
            pl.BlockSpec((BN, DE), lambda i: (i, 0)),
        ],
        out_shape=[
            jax.ShapeDtypeStruct((N, D), jnp.float32),
            jax.ShapeDtypeStruct((N, D // 2), jnp.int32),
            jax.ShapeDtypeStruct((N, D // 2), jnp.int32),
            jax.ShapeDtypeStruct((N, DE), jnp.float32),
        ],
    )(x, sc, WlT, WrT, bl2, br2, WeT, A_att, g1, b1)


BE = 2048


def _p4_body(ea_ref, gl_ref, gr_ref, we_ref, aatt_ref, e8_ref,
             y_ref, exz_ref):
    gl = _unpack2(gl_ref[...])
    gr = _unpack2(gr_ref[...])
    ee = jnp.dot(ea_ref[...], we_ref[...], preferred_element_type=jnp.float32)
    m = gl + gr + ee
    m = jnp.where(m > 0, m, 0.2 * m)
    a = jnp.dot(m, aatt_ref[...], preferred_element_type=jnp.float32)
    ex = jnp.exp(a)
    exz_ref[...] = jnp.concatenate([ex, jnp.zeros_like(ex)], axis=1)
    y = jnp.dot(ex, e8_ref[...], preferred_element_type=jnp.float32) * gl
    y_ref[...] = jnp.stack([y[:, : D // 2], y[:, D // 2 :]])


def _p4_call(ea, gl, gr, WeT, A_att, E8):
    nb = EP // BE
    full = lambda i: (0, 0)
    return pl.pallas_call(
        _p4_body,
        grid=(nb,),
        in_specs=[
            pl.BlockSpec((BE, DE), lambda i: (i, 0)),
            pl.BlockSpec((BE, D // 2), lambda i: (i, 0)),
            pl.BlockSpec((BE, D // 2), lambda i: (i, 0)),
            pl.BlockSpec((DE, D), full),
            pl.BlockSpec((D, H), full),
            pl.BlockSpec((H, D), full),
        ],
        out_specs=[
            pl.BlockSpec((NC, BE, D // 2), lambda i: (0, i, 0)),
            pl.BlockSpec((BE, DE), lambda i: (i, 0)),
        ],
        out_shape=[
            jax.ShapeDtypeStruct((NC, EP, D // 2), jnp.float32),
            jax.ShapeDtypeStruct((EP, DE), jnp.float32),
        ],
    )(ea, gl, gr, WeT, A_att, E8)


def _p6_body(x_ref, xl_ref, exl_ref, num_ref, den_ref, e8_ref, gb_ref,
             g2_ref, b2g_ref, w1_ref, b1f_ref, w2_ref, b2f_ref,
             out_ref):
    x = x_ref[...]
    xl = xl_ref[...]
    exl = exl_ref[...][:, :H]
    num = jnp.concatenate([num_ref[0], num_ref[1]], axis=1)
    e8 = e8_ref[...]
    num = num + jnp.dot(exl, e8, preferred_element_type=jnp.float32) * xl
    den = den_ref[0][:, :H] + den_ref[1][:, :H] + exl
    den256 = jnp.dot(den, e8, preferred_element_type=jnp.float32)
    sa = num / den256 + gb_ref[...]
    x1 = x + sa
    mu = jnp.mean(x1, axis=-1, keepdims=True)
    xc = x1 - mu
    var = jnp.mean(xc * xc, axis=-1, keepdims=True)
    h = xc / jnp.sqrt(var + 1e-5) * g2_ref[...] + b2g_ref[...]
    f = jnp.dot(h, w1_ref[...], preferred_element_type=jnp.float32) + b1f_ref[...]
    f = 0.5 * f * (1.0 + lax.erf(f * 0.7071067811865476))
    ff = jnp.dot(f, w2_ref[...], preferred_element_type=jnp.float32) + b2f_ref[...]
    out_ref[...] = x1 + ff


def _p6_call(x, xl, exl, num, den, E8, gb, g2, b2g, W1T, b1f, W2T, b2f):
    nb = N // BN
    full = lambda i: (0, 0)
    return pl.pallas_call(
        _p6_body,
        grid=(nb,),
        in_specs=[
            pl.BlockSpec((BN, D), lambda i: (i, 0)),
            pl.BlockSpec((BN, D), lambda i: (i, 0)),
            pl.BlockSpec((BN, DE), lambda i: (i, 0)),
            pl.BlockSpec((NC, BN, D // 2), lambda i: (0, i, 0)),
            pl.BlockSpec((NC, BN, DE), lambda i: (0, i, 0)),
            pl.BlockSpec((H, D), full),
            pl.BlockSpec((1, D), full),
            pl.BlockSpec((1, D), full),
            pl.BlockSpec((1, D), full),
            pl.BlockSpec((D, 2 * D), full),
            pl.BlockSpec((1, 2 * D), full),
            pl.BlockSpec((2 * D, D), full),
            pl.BlockSpec((1, D), full),
        ],
        out_specs=pl.BlockSpec((BN, D), lambda i: (i, 0)),
        out_shape=jax.ShapeDtypeStruct((N, D), jnp.float32),
    )(x, xl, exl, num, den, E8, gb, g2, b2g, W1T, b1f, W2T, b2f)


def kernel(x, edge_index, edge_attr, Wl, bl, Wr, br, We, att, gat_bias,
           ln1_g, ln1_b, ln2_g, ln2_b, W1, b1, W2, b2):
    pad = EP - E
    src = jnp.concatenate([edge_index[0], jnp.zeros((pad,), edge_index.dtype)])
    dst_g = jnp.concatenate([edge_index[1], jnp.zeros((pad,), edge_index.dtype)])
    dst = jnp.concatenate([edge_index[1], jnp.full((pad,), TRASH, edge_index.dtype)])
    ea_p = jnp.concatenate([edge_attr, jnp.zeros((pad, DE), edge_attr.dtype)])
    WlT = Wl.T
    WrT = Wr.T
    WeT = We.T
    W1T = W1.T
    W2T = W2.T
    A_att = (jnp.zeros((D, H), jnp.float32)
             .at[jnp.arange(D), jnp.arange(D) // C].set(att.reshape(-1)))
    E8 = (jnp.arange(D)[None, :] // C == jnp.arange(H)[:, None]).astype(jnp.float32)
    z128 = jnp.zeros((NPAD, D // 2), jnp.float32)
    r2 = lambda v: v.reshape(1, -1)

    sc = _p1_attr_sums(dst, ea_p, z128)
    xl, xlp, xrp, exl = _p2_call(x, sc, WlT, WrT, r2(bl), r2(br), WeT,
                                 A_att, r2(ln1_g), r2(ln1_b))
    gl, gr = _p3_gather(src, dst_g, xlp, xrp)
    y, exz = _p4_call(ea_p, gl.reshape(EP, D // 2), gr.reshape(EP, D // 2),
                      WeT, A_att, E8)
    num = _p5_scatter(dst.reshape(NS, ITER, CB), y, z128)
    den = _p5b_den_scatter(dst, exz, z128)
    out = _p6_call(x, xl, exl, num, den, E8, r2(gat_bias), r2(ln2_g), r2(ln2_b),
                   W1T, r2(b1), W2T, r2(b2))
    return out
